# Initial kernel scaffold; baseline (speedup 1.0000x reference)
import jax
import jax.numpy as jnp
from jax.experimental import pallas as pl


def kernel(xyz, feat, W, b, gamma, beta):
    raise NotImplementedError("write your pallas kernel here")


# speedup vs baseline: 4.1060x; 4.1060x over previous
import functools

import jax
import jax.numpy as jnp
from jax import lax
from jax.experimental import pallas as pl
from jax.experimental.pallas import tpu as pltpu
from jax.experimental.pallas import tpu_sc as plsc

_B, _N, _F, _K = 4, 1024, 1024, 8
_EPS = 1e-5


def _knn_body(xr_ref, xc_ref, out_ref):
    x0r = xr_ref[0:1, :]
    x1r = xr_ref[1:2, :]
    x0c = xc_ref[:, 0:1]
    x1c = xc_ref[:, 1:2]
    xxr = x0r * x0r + x1r * x1r
    xxc = x0c * x0c + x1c * x1c
    q = lambda v: v.astype(jnp.bfloat16).astype(jnp.float32)
    inner = 2.0 * (q(x0c) * q(x0r) + q(x1c) * q(x1r))
    d = (xxr - inner) + xxc
    iota = lax.broadcasted_iota(jnp.int32, (_N, _N), 1)
    big = jnp.float32(jnp.inf)
    cols = []
    for _ in range(_K):
        m = jnp.min(d, axis=1, keepdims=True)
        am = jnp.min(jnp.where(d == m, iota, _N), axis=1, keepdims=True)
        cols.append(am)
        d = jnp.where(iota == am, big, d)
    out_ref[...] = jnp.concatenate(cols, axis=1)


def _knn(xr, xc):
    return pl.pallas_call(
        _knn_body,
        grid=(_B,),
        in_specs=[
            pl.BlockSpec((None, 2, _N), lambda b: (b, 0, 0)),
            pl.BlockSpec((None, _N, 2), lambda b: (b, 0, 0)),
        ],
        out_specs=pl.BlockSpec((None, _N, _K), lambda b: (b, 0, 0)),
        out_shape=jax.ShapeDtypeStruct((_B, _N, _K), jnp.int32),
    )(xr, xc)


_NC, _NS, _L = 2, 16, 16
_NW = _NC * _NS
_ROWS_PER_W = (_B * _N) // _NW
_CH = 8
_G = _CH * _K
_NCHUNK = _ROWS_PER_W // _CH


def _gm_body(feat_t_hbm, cidx_hbm, out_hbm, idx_v, rows_v, acc_v, sem):
    wid = lax.axis_index("s") * _NC + lax.axis_index("c")
    row0 = wid * _ROWS_PER_W

    def chunk_body(ch, carry):
        base = row0 + ch * _CH
        pltpu.sync_copy(cidx_hbm.at[pl.ds(base * _K, _G)], idx_v)
        pltpu.async_copy(feat_t_hbm.at[idx_v], rows_v, sem).wait()

        def lane_body(i, c2):
            off = pl.multiple_of(i * _L, _L)
            for o in range(_CH):
                s = rows_v[o * _K, pl.ds(off, _L)]
                for j in range(1, _K):
                    s = s + rows_v[o * _K + j, pl.ds(off, _L)]
                acc_v[o, pl.ds(off, _L)] = s * jnp.float32(1.0 / _K)
            return c2

        lax.fori_loop(0, _F // _L, lane_body, 0)
        pltpu.sync_copy(acc_v, out_hbm.at[pl.ds(base, _CH)])
        return carry

    lax.fori_loop(0, _NCHUNK, chunk_body, 0)


def _gather_mean(feat_t_all, cidx_flat):
    mesh = plsc.VectorSubcoreMesh(core_axis_name="c", subcore_axis_name="s")
    fn = functools.partial(
        pl.kernel,
        out_type=jax.ShapeDtypeStruct((_B * _N, _F), jnp.float32),
        mesh=mesh,
        scratch_types=[
            pltpu.VMEM((_G,), jnp.int32),
            pltpu.VMEM((_G, _F), jnp.float32),
            pltpu.VMEM((_CH, _F), jnp.float32),
            pltpu.SemaphoreType.DMA,
        ],
    )(_gm_body)
    return fn(feat_t_all, cidx_flat)


def _mm_body(feat_ref, m_ref, wt_ref, b_ref, t_ref, s1_ref, s2_ref):
    lap = feat_ref[...] - m_ref[...]
    t = jnp.dot(
        lap,
        wt_ref[...],
        preferred_element_type=jnp.float32,
        precision=lax.Precision.HIGHEST,
    )
    t = t + b_ref[...]
    t_ref[...] = t
    s1_ref[...] = jnp.sum(t, axis=1, keepdims=True)
    s2_ref[...] = jnp.sum(t * t, axis=1, keepdims=True)


def _matmul_stats(feat, m, wt, bias2):
    return pl.pallas_call(
        _mm_body,
        grid=(_B,),
        in_specs=[
            pl.BlockSpec((None, _N, _F), lambda b: (b, 0, 0)),
            pl.BlockSpec((None, _N, _F), lambda b: (b, 0, 0)),
            pl.BlockSpec((_F, _F), lambda b: (0, 0)),
            pl.BlockSpec((1, _F), lambda b: (0, 0)),
        ],
        out_specs=[
            pl.BlockSpec((None, _N, _F), lambda b: (b, 0, 0)),
            pl.BlockSpec((None, _N, 1), lambda b: (b, 0, 0)),
            pl.BlockSpec((None, _N, 1), lambda b: (b, 0, 0)),
        ],
        out_shape=[
            jax.ShapeDtypeStruct((_B, _N, _F), jnp.float32),
            jax.ShapeDtypeStruct((_B, _N, 1), jnp.float32),
            jax.ShapeDtypeStruct((_B, _N, 1), jnp.float32),
        ],
    )(feat, m, wt, bias2)


def _bn_body(t_ref, feat_ref, s1_ref, s2_ref, g_ref, bt_ref, out_ref):
    inv = jnp.float32(1.0 / (_B * _F))
    s1 = s1_ref[0] + s1_ref[1] + s1_ref[2] + s1_ref[3]
    s2 = s2_ref[0] + s2_ref[1] + s2_ref[2] + s2_ref[3]
    mean = s1 * inv
    var = s2 * inv - mean * mean
    rstd = lax.rsqrt(var + jnp.float32(_EPS))
    xn = (t_ref[...] - mean) * rstd
    y = xn * g_ref[...] + bt_ref[...]
    out_ref[...] = feat_ref[...] + jnp.maximum(y, 0.0)


def _bn_finish(t, feat, s1, s2, gamma2, beta2):
    return pl.pallas_call(
        _bn_body,
        grid=(_B,),
        in_specs=[
            pl.BlockSpec((None, _N, _F), lambda b: (b, 0, 0)),
            pl.BlockSpec((None, _N, _F), lambda b: (b, 0, 0)),
            pl.BlockSpec((_B, _N, 1), lambda b: (0, 0, 0)),
            pl.BlockSpec((_B, _N, 1), lambda b: (0, 0, 0)),
            pl.BlockSpec((_N, 1), lambda b: (0, 0)),
            pl.BlockSpec((_N, 1), lambda b: (0, 0)),
        ],
        out_specs=pl.BlockSpec((None, _N, _F), lambda b: (b, 0, 0)),
        out_shape=jax.ShapeDtypeStruct((_B, _N, _F), jnp.float32),
    )(t, feat, s1, s2, gamma2, beta2)


def kernel(xyz, feat, W, b, gamma, beta):
    x2 = xyz[:, :, 0:2]
    xr = jnp.transpose(x2, (0, 2, 1))
    idx = _knn(xr, x2)

    idxr = idx.reshape(_B, _K, _N // _K, _K)
    c = jnp.transpose(idxr, (0, 2, 3, 1)).reshape(_B, _N, _K)
    cglob = c + (jnp.arange(_B, dtype=jnp.int32) * _N)[:, None, None]
    cidx_flat = cglob.reshape(_B * _N * _K)

    feat_t_all = jnp.transpose(feat, (0, 2, 1)).reshape(_B * _N, _F)
    m = _gather_mean(feat_t_all, cidx_flat).reshape(_B, _N, _F)

    wt = jnp.transpose(W)
    t, s1, s2 = _matmul_stats(feat, m, wt, b.reshape(1, _F))
    return _bn_finish(t, feat, s1, s2, gamma.reshape(_N, 1), beta.reshape(_N, 1))
